# Initial kernel scaffold; baseline (speedup 1.0000x reference)
import jax
import jax.numpy as jnp
from jax.experimental import pallas as pl


def kernel(x, Wr, br, w1, w2, w3):
    raise NotImplementedError("write your pallas kernel here")


# speedup vs baseline: 1.2206x; 1.2206x over previous
import jax
import jax.numpy as jnp
from jax import lax
from jax.experimental import pallas as pl
from jax.experimental.pallas import tpu as pltpu
from jax.experimental.pallas import tpu_sc as plsc

D = 768
H = 1024
E = 8
T = 2048
CAP = 512
NSLOT = E * CAP
TRASH = NSLOT
NC, NS = 2, 16
NW = NC * NS
CHUNK = 64
BLK = 128


def _router_body(x_ref, wr_ref, br_ref, coef_ref, idx_ref):
    x = x_ref[...]
    logits = jnp.dot(x, wr_ref[...], preferred_element_type=jnp.float32)
    logits = logits + br_ref[...]
    eidx = lax.broadcasted_iota(jnp.int32, (T, E), 1)

    m1 = jnp.max(logits, axis=1, keepdims=True)
    a1 = jnp.min(jnp.where(logits == m1, eidx, E), axis=1, keepdims=True)
    oh0 = eidx == a1
    masked = jnp.where(oh0, -jnp.inf, logits)
    m2 = jnp.max(masked, axis=1, keepdims=True)
    a2 = jnp.min(jnp.where(masked == m2, eidx, E), axis=1, keepdims=True)
    oh1 = eidx == a2

    dexp = jnp.exp(m2 - m1)
    s0 = 1.0 / (1.0 + dexp)
    s1 = dexp / (1.0 + dexp)

    oh0f = oh0.astype(jnp.float32)
    oh1f = oh1.astype(jnp.float32)
    both = jnp.concatenate([oh0f, oh0f + oh1f], axis=1)
    r = lax.broadcasted_iota(jnp.int32, (BLK, BLK), 0)
    c = lax.broadcasted_iota(jnp.int32, (BLK, BLK), 1)
    tri = (c <= r).astype(jnp.float32)
    run = jnp.zeros((1, 2 * E), jnp.float32)
    pieces = []
    for b in range(T // BLK):
        blk = both[b * BLK:(b + 1) * BLK, :]
        wcs = jnp.dot(tri, blk, preferred_element_type=jnp.float32) + run
        run = wcs[BLK - 1:BLK, :]
        pieces.append(wcs)
    csum = jnp.concatenate(pieces, axis=0)

    pos0 = jnp.sum(oh0f * csum[:, :E], axis=1, keepdims=True)
    pos1 = jnp.sum(oh1f * csum[:, E:], axis=1, keepdims=True)
    keep0 = pos0 < CAP
    keep1 = pos1 < CAP
    k0f = keep0.astype(jnp.float32)
    k1f = keep1.astype(jnp.float32)

    c0 = s0 * k0f
    c1 = s1 * k1f
    cx = s0 * (1.0 - k0f) + s1 * (1.0 - k1f)
    zf = jnp.zeros((T, 1), jnp.float32)
    coef_ref[...] = jnp.concatenate(
        [c0, c1, cx, s0, s1, zf, zf, zf], axis=1)

    pos0i = pos0.astype(jnp.int32)
    pos1i = pos1.astype(jnp.int32)
    slot0 = a1 * CAP + pos0i
    slot1 = a2 * CAP + pos1i
    dst0 = jnp.where(keep0, slot0, TRASH)
    dst1 = jnp.where(keep1, slot1, TRASH)
    src0 = jnp.where(keep0, slot0, 0)
    src1 = jnp.where(keep1, slot1, 0)
    zi = jnp.zeros((T, 1), jnp.int32)
    idx_ref[...] = jnp.concatenate(
        [dst0, dst1, src0, src1, pos0i, pos1i, zi, zi], axis=1)


def _router(x_flat, Wr, br2d):
    return pl.pallas_call(
        _router_body,
        out_shape=(
            jax.ShapeDtypeStruct((T, 8), jnp.float32),
            jax.ShapeDtypeStruct((T, 8), jnp.int32),
        ),
    )(x_flat, Wr, br2d)


def _dispatch_body(x_hbm, dst_hbm, zeros_hbm, bigA, bigB,
                   rows_v, idx_v, midx_v, sem):
    cid = lax.axis_index("c")
    sid = lax.axis_index("s")
    pltpu.sync_copy(zeros_hbm, rows_v)
    zbase = cid * (NSLOT // NC) + sid * (NSLOT // NW)
    pltpu.sync_copy(rows_v, bigA.at[pl.ds(zbase, CHUNK)])
    pltpu.sync_copy(rows_v, bigA.at[pl.ds(zbase + CHUNK, CHUNK)])
    pltpu.sync_copy(rows_v, bigB.at[pl.ds(zbase, CHUNK)])
    pltpu.sync_copy(rows_v, bigB.at[pl.ds(zbase + CHUNK, CHUNK)])
    plsc.subcore_barrier()
    lo = cid * (NSLOT // NC)
    hi = lo + NSLOT // NC
    for half in range(2):
        tb = sid * 2 * CHUNK + half * CHUNK
        pltpu.sync_copy(x_hbm.at[pl.ds(tb, CHUNK)], rows_v)
        for k in range(2):
            pltpu.sync_copy(dst_hbm.at[k, pl.ds(tb, CHUNK)], idx_v)
            for j in range(CHUNK // 16):
                v = idx_v[pl.ds(16 * j, 16)]
                mine = jnp.logical_and(v >= lo, v < hi)
                midx_v[pl.ds(16 * j, 16)] = jnp.where(mine, v, TRASH)
            big = bigA if k == 0 else bigB
            pltpu.async_copy(rows_v, big.at[midx_v], sem).wait()


def _dispatch(x_flat, dsts, zeros):
    mesh = plsc.VectorSubcoreMesh(core_axis_name="c", subcore_axis_name="s")
    return pl.kernel(
        _dispatch_body,
        out_type=(
            jax.ShapeDtypeStruct((NSLOT + 1, D), jnp.float32),
            jax.ShapeDtypeStruct((NSLOT + 1, D), jnp.float32),
        ),
        mesh=mesh,
        scratch_types=[
            pltpu.VMEM((CHUNK, D), jnp.float32),
            pltpu.VMEM((CHUNK,), jnp.int32),
            pltpu.VMEM((CHUNK,), jnp.int32),
            pltpu.SemaphoreType.DMA,
        ],
    )(x_flat, dsts, zeros)


def _ff_body(bigA_ref, bigB_ref, w1_ref, w2_ref, w3_ref, out_ref):
    xg = bigA_ref[...] + bigB_ref[...]
    h = (jnp.dot(xg, w2_ref[0], preferred_element_type=jnp.float32)
         * jnp.dot(xg, w1_ref[0], preferred_element_type=jnp.float32))
    h = jax.nn.gelu(h)
    out_ref[...] = jnp.dot(h, w3_ref[0], preferred_element_type=jnp.float32)


def _ff(bigA, bigB, w1, w2, w3):
    return pl.pallas_call(
        _ff_body,
        grid=(E,),
        in_specs=[
            pl.BlockSpec((CAP, D), lambda e: (e, 0)),
            pl.BlockSpec((CAP, D), lambda e: (e, 0)),
            pl.BlockSpec((1, D, H), lambda e: (e, 0, 0)),
            pl.BlockSpec((1, D, H), lambda e: (e, 0, 0)),
            pl.BlockSpec((1, H, D), lambda e: (e, 0, 0)),
        ],
        out_specs=pl.BlockSpec((CAP, D), lambda e: (e, 0)),
        out_shape=jax.ShapeDtypeStruct((NSLOT + 1, D), jnp.float32),
        input_output_aliases={0: 0},
    )(bigA, bigB, w1, w2, w3)


def _combine_body(big_hbm, src_hbm, g0_hbm, g1_hbm, idx_v, rows_v, sem):
    wid = lax.axis_index("s") * NC + lax.axis_index("c")
    base = wid * CHUNK
    pltpu.sync_copy(src_hbm.at[0, wid], idx_v)
    pltpu.async_copy(big_hbm.at[idx_v], rows_v, sem).wait()
    pltpu.sync_copy(rows_v, g0_hbm.at[pl.ds(base, CHUNK)])
    pltpu.sync_copy(src_hbm.at[1, wid], idx_v)
    pltpu.async_copy(big_hbm.at[idx_v], rows_v, sem).wait()
    pltpu.sync_copy(rows_v, g1_hbm.at[pl.ds(base, CHUNK)])


def _combine(big, srcs3):
    mesh = plsc.VectorSubcoreMesh(core_axis_name="c", subcore_axis_name="s")
    return pl.kernel(
        _combine_body,
        out_type=(
            jax.ShapeDtypeStruct((T, D), jnp.float32),
            jax.ShapeDtypeStruct((T, D), jnp.float32),
        ),
        mesh=mesh,
        scratch_types=[
            pltpu.VMEM((CHUNK,), jnp.int32),
            pltpu.VMEM((CHUNK, D), jnp.float32),
            pltpu.SemaphoreType.DMA,
        ],
    )(big, srcs3)


def _blend_body(g0_ref, g1_ref, x_ref, coef_ref, out_ref):
    c0 = coef_ref[:, 0:1]
    c1 = coef_ref[:, 1:2]
    cx = coef_ref[:, 2:3]
    out_ref[...] = c0 * g0_ref[...] + c1 * g1_ref[...] + cx * x_ref[...]


def _blend(g0, g1, x_flat, coef):
    return pl.pallas_call(
        _blend_body,
        out_shape=jax.ShapeDtypeStruct((T, D), jnp.float32),
    )(g0, g1, x_flat, coef)


def kernel(x, Wr, br, w1, w2, w3):
    x_flat = x.reshape(T, D)
    coef, idx = _router(x_flat, Wr, br.reshape(1, E))
    dsts = idx[:, 0:2].T.copy()
    srcs3 = idx[:, 2:4].T.reshape(2, NW, CHUNK)
    zeros = jnp.zeros((CHUNK, D), jnp.float32)
    bigA, bigB = _dispatch(x_flat, dsts, zeros)
    big = _ff(bigA, bigB, w1, w2, w3)
    g0, g1 = _combine(big, srcs3)
    out = _blend(g0, g1, x_flat, coef)
    return out.reshape(1, T, D)


# speedup vs baseline: 2.3912x; 1.9591x over previous
import jax
import jax.numpy as jnp
from jax import lax
from jax.experimental import pallas as pl
from jax.experimental.pallas import tpu as pltpu
from jax.experimental.pallas import tpu_sc as plsc

D = 768
H = 1024
E = 8
T = 2048
CAP = 512
NSLOT = E * CAP
NC, NS = 2, 16
NW = NC * NS
NPLANE = NSLOT + NS
TPS = T // NS
CHUNK = 64
BLK = 128


def _router_body(x_ref, wr_ref, br_ref, coef_ref, idx_ref):
    x = x_ref[...]
    logits = jnp.dot(x, wr_ref[...], preferred_element_type=jnp.float32)
    logits = logits + br_ref[...]
    eidx = lax.broadcasted_iota(jnp.int32, (T, E), 1)

    m1 = jnp.max(logits, axis=1, keepdims=True)
    a1 = jnp.min(jnp.where(logits == m1, eidx, E), axis=1, keepdims=True)
    oh0 = eidx == a1
    masked = jnp.where(oh0, -jnp.inf, logits)
    m2 = jnp.max(masked, axis=1, keepdims=True)
    a2 = jnp.min(jnp.where(masked == m2, eidx, E), axis=1, keepdims=True)
    oh1 = eidx == a2

    dexp = jnp.exp(m2 - m1)
    s0 = 1.0 / (1.0 + dexp)
    s1 = dexp / (1.0 + dexp)

    oh0f = oh0.astype(jnp.float32)
    oh1f = oh1.astype(jnp.float32)
    both = jnp.concatenate([oh0f, oh0f + oh1f], axis=1)
    r = lax.broadcasted_iota(jnp.int32, (BLK, BLK), 0)
    c = lax.broadcasted_iota(jnp.int32, (BLK, BLK), 1)
    tri = (c <= r).astype(jnp.float32)
    run = jnp.zeros((1, 2 * E), jnp.float32)
    pieces = []
    for b in range(T // BLK):
        blk = both[b * BLK:(b + 1) * BLK, :]
        wcs = jnp.dot(tri, blk, preferred_element_type=jnp.float32) + run
        run = wcs[BLK - 1:BLK, :]
        pieces.append(wcs)
    csum = jnp.concatenate(pieces, axis=0)

    pos0 = jnp.sum(oh0f * csum[:, :E], axis=1, keepdims=True)
    pos1 = jnp.sum(oh1f * csum[:, E:], axis=1, keepdims=True)
    keep0 = pos0 < CAP
    keep1 = pos1 < CAP
    k0f = keep0.astype(jnp.float32)
    k1f = keep1.astype(jnp.float32)

    c0 = s0 * k0f
    c1 = s1 * k1f
    cx = s0 * (1.0 - k0f) + s1 * (1.0 - k1f)
    zf = jnp.zeros((T, 1), jnp.float32)
    coef_ref[...] = jnp.concatenate(
        [c0, c1, cx, s0, s1, zf, zf, zf], axis=1)

    pos0i = pos0.astype(jnp.int32)
    pos1i = pos1.astype(jnp.int32)
    slot0 = a1 * CAP + pos0i
    slot1 = a2 * CAP + pos1i
    tidx = lax.broadcasted_iota(jnp.int32, (T, 1), 0)
    trash = NSLOT + jnp.right_shift(tidx, 7)
    dst0 = jnp.where(keep0, slot0, trash)
    dst1 = jnp.where(keep1, slot1, trash)
    src0 = jnp.where(keep0, slot0, 0)
    src1 = jnp.where(keep1, slot1, 0)
    zi = jnp.zeros((T, 1), jnp.int32)
    idx_ref[...] = jnp.concatenate(
        [dst0, dst1, src0, src1, pos0i, pos1i, zi, zi], axis=1)


def _router(x_flat, Wr, br2d):
    return pl.pallas_call(
        _router_body,
        out_shape=(
            jax.ShapeDtypeStruct((T, 8), jnp.float32),
            jax.ShapeDtypeStruct((T, 8), jnp.int32),
        ),
    )(x_flat, Wr, br2d)


def _dispatch_body(x_hbm, dst_hbm, zeros_hbm, big,
                   rows_a, rows_b, idx_a, idx_b, sem):
    cid = lax.axis_index("c")
    sid = lax.axis_index("s")
    plane = big.at[cid]
    pltpu.sync_copy(zeros_hbm, rows_a)
    zbase = sid * (NSLOT // NS)
    zcps = [pltpu.async_copy(rows_a, plane.at[pl.ds(zbase + i * CHUNK, CHUNK)],
                             sem) for i in range(NSLOT // NS // CHUNK)]
    for cp in zcps:
        cp.wait()
    plsc.subcore_barrier()
    tb = sid * TPS
    pltpu.sync_copy(dst_hbm.at[cid, pl.ds(tb, CHUNK)], idx_a)
    pltpu.sync_copy(dst_hbm.at[cid, pl.ds(tb + CHUNK, CHUNK)], idx_b)
    pltpu.sync_copy(x_hbm.at[pl.ds(tb, CHUNK)], rows_a)
    cp_a = pltpu.async_copy(rows_a, plane.at[idx_a], sem)
    pltpu.sync_copy(x_hbm.at[pl.ds(tb + CHUNK, CHUNK)], rows_b)
    cp_b = pltpu.async_copy(rows_b, plane.at[idx_b], sem)
    cp_a.wait()
    cp_b.wait()


def _dispatch(x_flat, dsts, zeros):
    mesh = plsc.VectorSubcoreMesh(core_axis_name="c", subcore_axis_name="s")
    return pl.kernel(
        _dispatch_body,
        out_type=jax.ShapeDtypeStruct((NC, NPLANE, D), jnp.float32),
        mesh=mesh,
        scratch_types=[
            pltpu.VMEM((CHUNK, D), jnp.float32),
            pltpu.VMEM((CHUNK, D), jnp.float32),
            pltpu.VMEM((CHUNK,), jnp.int32),
            pltpu.VMEM((CHUNK,), jnp.int32),
            pltpu.SemaphoreType.DMA,
        ],
    )(x_flat, dsts, zeros)


def _ff_body(big_ref, w1_ref, w2_ref, w3_ref, out_ref):
    xg = big_ref[0] + big_ref[1]
    h = (jnp.dot(xg, w2_ref[0], preferred_element_type=jnp.float32)
         * jnp.dot(xg, w1_ref[0], preferred_element_type=jnp.float32))
    h = jax.nn.gelu(h)
    out_ref[0] = jnp.dot(h, w3_ref[0], preferred_element_type=jnp.float32)


def _ff(big, w1, w2, w3):
    return pl.pallas_call(
        _ff_body,
        grid=(E,),
        in_specs=[
            pl.BlockSpec((NC, CAP, D), lambda e: (0, e, 0)),
            pl.BlockSpec((1, D, H), lambda e: (e, 0, 0)),
            pl.BlockSpec((1, D, H), lambda e: (e, 0, 0)),
            pl.BlockSpec((1, H, D), lambda e: (e, 0, 0)),
        ],
        out_specs=pl.BlockSpec((1, CAP, D), lambda e: (0, e, 0)),
        out_shape=jax.ShapeDtypeStruct((NC, NPLANE, D), jnp.float32),
        input_output_aliases={0: 0},
    )(big, w1, w2, w3)


def _combine_body(big_hbm, src_hbm, g0_hbm, g1_hbm, idx_v, rows_v, sem):
    wid = lax.axis_index("s") * NC + lax.axis_index("c")
    base = wid * CHUNK
    pltpu.sync_copy(src_hbm.at[wid], idx_v)
    pltpu.async_copy(big_hbm.at[idx_v], rows_v, sem).wait()
    pltpu.sync_copy(rows_v.at[pl.ds(0, CHUNK)], g0_hbm.at[pl.ds(base, CHUNK)])
    pltpu.sync_copy(rows_v.at[pl.ds(CHUNK, CHUNK)],
                    g1_hbm.at[pl.ds(base, CHUNK)])


def _combine(big_flat, srcs2):
    mesh = plsc.VectorSubcoreMesh(core_axis_name="c", subcore_axis_name="s")
    return pl.kernel(
        _combine_body,
        out_type=(
            jax.ShapeDtypeStruct((T, D), jnp.float32),
            jax.ShapeDtypeStruct((T, D), jnp.float32),
        ),
        mesh=mesh,
        scratch_types=[
            pltpu.VMEM((2 * CHUNK,), jnp.int32),
            pltpu.VMEM((2 * CHUNK, D), jnp.float32),
            pltpu.SemaphoreType.DMA,
        ],
    )(big_flat, srcs2)


def _blend_body(g0_ref, g1_ref, x_ref, coef_ref, out_ref):
    c0 = coef_ref[:, 0:1]
    c1 = coef_ref[:, 1:2]
    cx = coef_ref[:, 2:3]
    out_ref[...] = c0 * g0_ref[...] + c1 * g1_ref[...] + cx * x_ref[...]


def _blend(g0, g1, x_flat, coef):
    return pl.pallas_call(
        _blend_body,
        out_shape=jax.ShapeDtypeStruct((T, D), jnp.float32),
    )(g0, g1, x_flat, coef)


def kernel(x, Wr, br, w1, w2, w3):
    x_flat = x.reshape(T, D)
    coef, idx = _router(x_flat, Wr, br.reshape(1, E))
    dsts = idx[:, 0:2].T.copy()
    src0 = idx[:, 2].reshape(NW, CHUNK)
    src1 = idx[:, 3].reshape(NW, CHUNK)
    srcs2 = jnp.concatenate([src0, src1], axis=1)
    zeros = jnp.zeros((CHUNK, D), jnp.float32)
    big = _dispatch(x_flat, dsts, zeros)
    big = _ff(big, w1, w2, w3)
    g0, g1 = _combine(big.reshape(NC * NPLANE, D), srcs2)
    out = _blend(g0, g1, x_flat, coef)
    return out.reshape(1, T, D)


# speedup vs baseline: 2.3988x; 1.0032x over previous
import jax
import jax.numpy as jnp
from jax import lax
from jax.experimental import pallas as pl
from jax.experimental.pallas import tpu as pltpu
from jax.experimental.pallas import tpu_sc as plsc

D = 768
H = 1024
E = 8
T = 2048
CAP = 512
NSLOT = E * CAP
NC, NS = 2, 16
NW = NC * NS
NPLANE = NSLOT + NS
TPS = T // NS
CHUNK = 64
BLK = 128


def _router_body(x_ref, wr_ref, br_ref, coef_ref, idx_ref):
    x = x_ref[...]
    logits = jnp.dot(x, wr_ref[...], preferred_element_type=jnp.float32)
    logits = logits + br_ref[...]
    eidx = lax.broadcasted_iota(jnp.int32, (T, E), 1)

    m1 = jnp.max(logits, axis=1, keepdims=True)
    a1 = jnp.min(jnp.where(logits == m1, eidx, E), axis=1, keepdims=True)
    oh0 = eidx == a1
    masked = jnp.where(oh0, -jnp.inf, logits)
    m2 = jnp.max(masked, axis=1, keepdims=True)
    a2 = jnp.min(jnp.where(masked == m2, eidx, E), axis=1, keepdims=True)
    oh1 = eidx == a2

    dexp = jnp.exp(m2 - m1)
    s0 = 1.0 / (1.0 + dexp)
    s1 = dexp / (1.0 + dexp)

    oh0f = oh0.astype(jnp.float32)
    oh1f = oh1.astype(jnp.float32)
    both = jnp.concatenate([oh0f, oh0f + oh1f], axis=1)
    r = lax.broadcasted_iota(jnp.int32, (BLK, BLK), 0)
    c = lax.broadcasted_iota(jnp.int32, (BLK, BLK), 1)
    tri = (c <= r).astype(jnp.float32)
    run = jnp.zeros((1, 2 * E), jnp.float32)
    pieces = []
    for b in range(T // BLK):
        blk = both[b * BLK:(b + 1) * BLK, :]
        wcs = jnp.dot(tri, blk, preferred_element_type=jnp.float32) + run
        run = wcs[BLK - 1:BLK, :]
        pieces.append(wcs)
    csum = jnp.concatenate(pieces, axis=0)

    pos0 = jnp.sum(oh0f * csum[:, :E], axis=1, keepdims=True)
    pos1 = jnp.sum(oh1f * csum[:, E:], axis=1, keepdims=True)
    keep0 = pos0 < CAP
    keep1 = pos1 < CAP
    k0f = keep0.astype(jnp.float32)
    k1f = keep1.astype(jnp.float32)

    c0 = s0 * k0f
    c1 = s1 * k1f
    cx = s0 * (1.0 - k0f) + s1 * (1.0 - k1f)
    zf = jnp.zeros((T, 1), jnp.float32)
    coef_ref[...] = jnp.concatenate(
        [c0, c1, cx, s0, s1, zf, zf, zf], axis=1)

    pos0i = pos0.astype(jnp.int32)
    pos1i = pos1.astype(jnp.int32)
    slot0 = a1 * CAP + pos0i
    slot1 = a2 * CAP + pos1i
    tidx = lax.broadcasted_iota(jnp.int32, (T, 1), 0)
    trash = NSLOT + jnp.right_shift(tidx, 7)
    dst0 = jnp.where(keep0, slot0, trash)
    dst1 = jnp.where(keep1, slot1, trash)
    src0 = jnp.where(keep0, slot0, 0)
    src1 = jnp.where(keep1, slot1, 0)
    zi = jnp.zeros((T, 1), jnp.int32)
    idx_ref[...] = jnp.concatenate(
        [dst0, dst1, src0, src1, pos0i, pos1i, zi, zi], axis=1)


def _router(x_flat, Wr, br2d):
    return pl.pallas_call(
        _router_body,
        out_shape=(
            jax.ShapeDtypeStruct((T, 8), jnp.float32),
            jax.ShapeDtypeStruct((T, 8), jnp.int32),
        ),
    )(x_flat, Wr, br2d)


def _dispatch_body(x_hbm, dst_hbm, zeros_hbm, big,
                   rows_a, rows_b, idx_a, idx_b, sem):
    cid = lax.axis_index("c")
    sid = lax.axis_index("s")
    plane = big.at[cid]
    pltpu.sync_copy(zeros_hbm, rows_a)
    zbase = sid * (NSLOT // NS)
    zcps = [pltpu.async_copy(rows_a, plane.at[pl.ds(zbase + i * CHUNK, CHUNK)],
                             sem) for i in range(NSLOT // NS // CHUNK)]
    for cp in zcps:
        cp.wait()
    plsc.subcore_barrier()
    tb = sid * TPS
    pltpu.sync_copy(dst_hbm.at[cid, pl.ds(tb, CHUNK)], idx_a)
    pltpu.sync_copy(dst_hbm.at[cid, pl.ds(tb + CHUNK, CHUNK)], idx_b)
    pltpu.sync_copy(x_hbm.at[pl.ds(tb, CHUNK)], rows_a)
    cp_a = pltpu.async_copy(rows_a, plane.at[idx_a], sem)
    pltpu.sync_copy(x_hbm.at[pl.ds(tb + CHUNK, CHUNK)], rows_b)
    cp_b = pltpu.async_copy(rows_b, plane.at[idx_b], sem)
    cp_a.wait()
    cp_b.wait()


def _dispatch(x_flat, dsts, zeros):
    mesh = plsc.VectorSubcoreMesh(core_axis_name="c", subcore_axis_name="s")
    return pl.kernel(
        _dispatch_body,
        out_type=jax.ShapeDtypeStruct((NC, NPLANE, D), jnp.float32),
        mesh=mesh,
        scratch_types=[
            pltpu.VMEM((CHUNK, D), jnp.float32),
            pltpu.VMEM((CHUNK, D), jnp.float32),
            pltpu.VMEM((CHUNK,), jnp.int32),
            pltpu.VMEM((CHUNK,), jnp.int32),
            pltpu.SemaphoreType.DMA,
        ],
    )(x_flat, dsts, zeros)


def _ff_body(big_ref, w1_ref, w2_ref, w3_ref, out_ref):
    xg = (big_ref[0] + big_ref[1]).astype(jnp.bfloat16)
    h = (jnp.dot(xg, w2_ref[0].astype(jnp.bfloat16),
                 preferred_element_type=jnp.float32)
         * jnp.dot(xg, w1_ref[0].astype(jnp.bfloat16),
                   preferred_element_type=jnp.float32))
    h = jax.nn.gelu(h)
    out_ref[0] = jnp.dot(h.astype(jnp.bfloat16),
                         w3_ref[0].astype(jnp.bfloat16),
                         preferred_element_type=jnp.float32)


def _ff(big, w1, w2, w3):
    return pl.pallas_call(
        _ff_body,
        grid=(E,),
        in_specs=[
            pl.BlockSpec((NC, CAP, D), lambda e: (0, e, 0)),
            pl.BlockSpec((1, D, H), lambda e: (e, 0, 0)),
            pl.BlockSpec((1, D, H), lambda e: (e, 0, 0)),
            pl.BlockSpec((1, H, D), lambda e: (e, 0, 0)),
        ],
        out_specs=pl.BlockSpec((1, CAP, D), lambda e: (0, e, 0)),
        out_shape=jax.ShapeDtypeStruct((NC, NPLANE, D), jnp.float32),
        input_output_aliases={0: 0},
    )(big, w1, w2, w3)


def _combine_body(big_hbm, src_hbm, g0_hbm, g1_hbm, idx_v, rows_v, sem):
    wid = lax.axis_index("s") * NC + lax.axis_index("c")
    base = wid * CHUNK
    pltpu.sync_copy(src_hbm.at[wid], idx_v)
    pltpu.async_copy(big_hbm.at[idx_v], rows_v, sem).wait()
    pltpu.sync_copy(rows_v.at[pl.ds(0, CHUNK)], g0_hbm.at[pl.ds(base, CHUNK)])
    pltpu.sync_copy(rows_v.at[pl.ds(CHUNK, CHUNK)],
                    g1_hbm.at[pl.ds(base, CHUNK)])


def _combine(big_flat, srcs2):
    mesh = plsc.VectorSubcoreMesh(core_axis_name="c", subcore_axis_name="s")
    return pl.kernel(
        _combine_body,
        out_type=(
            jax.ShapeDtypeStruct((T, D), jnp.float32),
            jax.ShapeDtypeStruct((T, D), jnp.float32),
        ),
        mesh=mesh,
        scratch_types=[
            pltpu.VMEM((2 * CHUNK,), jnp.int32),
            pltpu.VMEM((2 * CHUNK, D), jnp.float32),
            pltpu.SemaphoreType.DMA,
        ],
    )(big_flat, srcs2)


def _blend_body(g0_ref, g1_ref, x_ref, coef_ref, out_ref):
    c0 = coef_ref[:, 0:1]
    c1 = coef_ref[:, 1:2]
    cx = coef_ref[:, 2:3]
    out_ref[...] = c0 * g0_ref[...] + c1 * g1_ref[...] + cx * x_ref[...]


def _blend(g0, g1, x_flat, coef):
    return pl.pallas_call(
        _blend_body,
        out_shape=jax.ShapeDtypeStruct((T, D), jnp.float32),
    )(g0, g1, x_flat, coef)


def kernel(x, Wr, br, w1, w2, w3):
    x_flat = x.reshape(T, D)
    coef, idx = _router(x_flat, Wr, br.reshape(1, E))
    dsts = idx[:, 0:2].T.copy()
    src0 = idx[:, 2].reshape(NW, CHUNK)
    src1 = idx[:, 3].reshape(NW, CHUNK)
    srcs2 = jnp.concatenate([src0, src1], axis=1)
    zeros = jnp.zeros((CHUNK, D), jnp.float32)
    big = _dispatch(x_flat, dsts, zeros)
    big = _ff(big, w1, w2, w3)
    g0, g1 = _combine(big.reshape(NC * NPLANE, D), srcs2)
    out = _blend(g0, g1, x_flat, coef)
    return out.reshape(1, T, D)


# speedup vs baseline: 2.6372x; 1.0994x over previous
import jax
import jax.numpy as jnp
from jax import lax
from jax.experimental import pallas as pl
from jax.experimental.pallas import tpu as pltpu
from jax.experimental.pallas import tpu_sc as plsc

D = 768
H = 1024
E = 8
T = 2048
CAP = 512
NSLOT = E * CAP
NC, NS = 2, 16
NW = NC * NS
NPLANE = NSLOT + NS
TPS = T // NS
CHUNK = 64
SPS = NSLOT // NS
BLK = 128


def _router_body(x_ref, wr_ref, br_ref, coef_ref, idx_ref):
    x = x_ref[...]
    logits = jnp.dot(x, wr_ref[...], preferred_element_type=jnp.float32)
    logits = logits + br_ref[...]
    eidx = lax.broadcasted_iota(jnp.int32, (T, E), 1)

    m1 = jnp.max(logits, axis=1, keepdims=True)
    a1 = jnp.min(jnp.where(logits == m1, eidx, E), axis=1, keepdims=True)
    oh0 = eidx == a1
    masked = jnp.where(oh0, -jnp.inf, logits)
    m2 = jnp.max(masked, axis=1, keepdims=True)
    a2 = jnp.min(jnp.where(masked == m2, eidx, E), axis=1, keepdims=True)
    oh1 = eidx == a2

    dexp = jnp.exp(m2 - m1)
    s0 = 1.0 / (1.0 + dexp)
    s1 = dexp / (1.0 + dexp)

    oh0f = oh0.astype(jnp.float32)
    oh1f = oh1.astype(jnp.float32)
    both = jnp.concatenate([oh0f, oh0f + oh1f], axis=1)
    r = lax.broadcasted_iota(jnp.int32, (BLK, BLK), 0)
    c = lax.broadcasted_iota(jnp.int32, (BLK, BLK), 1)
    tri = (c <= r).astype(jnp.float32)
    run = jnp.zeros((1, 2 * E), jnp.float32)
    pieces = []
    for b in range(T // BLK):
        blk = both[b * BLK:(b + 1) * BLK, :]
        wcs = jnp.dot(tri, blk, preferred_element_type=jnp.float32) + run
        run = wcs[BLK - 1:BLK, :]
        pieces.append(wcs)
    csum = jnp.concatenate(pieces, axis=0)

    pos0 = jnp.sum(oh0f * csum[:, :E], axis=1, keepdims=True)
    pos1 = jnp.sum(oh1f * csum[:, E:], axis=1, keepdims=True)
    keep0 = pos0 < CAP
    keep1 = pos1 < CAP
    k0f = keep0.astype(jnp.float32)
    k1f = keep1.astype(jnp.float32)

    c0 = s0 * k0f
    c1 = s1 * k1f
    cx = s0 * (1.0 - k0f) + s1 * (1.0 - k1f)
    zf = jnp.zeros((T, 1), jnp.float32)
    coef_ref[...] = jnp.concatenate(
        [c0, c1, cx, s0, s1, zf, zf, zf], axis=1)

    pos0i = pos0.astype(jnp.int32)
    pos1i = pos1.astype(jnp.int32)
    slot0 = a1 * CAP + pos0i
    slot1 = a2 * CAP + pos1i
    tidx = lax.broadcasted_iota(jnp.int32, (T, 1), 0)
    trash = NSLOT + jnp.right_shift(tidx, 7)
    dst0 = jnp.where(keep0, slot0, trash)
    dst1 = jnp.where(keep1, slot1, trash)
    src0 = jnp.where(keep0, slot0, 0)
    src1 = jnp.where(keep1, slot1, 0)
    zi = jnp.zeros((T, 1), jnp.int32)
    idx_ref[...] = jnp.concatenate(
        [dst0, dst1, src0, src1, pos0i, pos1i, zi, zi], axis=1)


def _router(x_flat, Wr, br2d):
    return pl.pallas_call(
        _router_body,
        out_shape=(
            jax.ShapeDtypeStruct((T, 8), jnp.float32),
            jax.ShapeDtypeStruct((T, 8), jnp.int32),
        ),
    )(x_flat, Wr, br2d)


def _dispatch_body(x_hbm, dst_hbm, big, vmap,
                   rows_a, rows_b, idx_a, idx_b, zv, ones_v, sem):
    cid = lax.axis_index("c")
    sid = lax.axis_index("s")
    plane = big.at[cid]
    mapp = vmap.at[cid]
    def _fill(i, _):
        for j in range(128 // 16):
            zv[i, pl.ds(16 * j, 16)] = jnp.zeros((16,), jnp.float32)
            ones_v[i, pl.ds(16 * j, 16)] = jnp.ones((16,), jnp.float32)
        return 0
    lax.fori_loop(0, CHUNK, _fill, 0)
    for j in range(SPS // CHUNK):
        pltpu.sync_copy(zv, mapp.at[pl.ds(sid * SPS + j * CHUNK, CHUNK)])
    plsc.subcore_barrier()
    tb = sid * TPS
    pltpu.sync_copy(dst_hbm.at[cid, pl.ds(tb, CHUNK)], idx_a)
    pltpu.sync_copy(dst_hbm.at[cid, pl.ds(tb + CHUNK, CHUNK)], idx_b)
    pltpu.sync_copy(x_hbm.at[pl.ds(tb, CHUNK)], rows_a)
    cp0 = pltpu.async_copy(rows_a, plane.at[idx_a], sem)
    cp1 = pltpu.async_copy(ones_v, mapp.at[idx_a], sem)
    pltpu.sync_copy(x_hbm.at[pl.ds(tb + CHUNK, CHUNK)], rows_b)
    cp2 = pltpu.async_copy(rows_b, plane.at[idx_b], sem)
    cp3 = pltpu.async_copy(ones_v, mapp.at[idx_b], sem)
    cp0.wait()
    cp1.wait()
    cp2.wait()
    cp3.wait()


def _dispatch(x_flat, dsts):
    mesh = plsc.VectorSubcoreMesh(core_axis_name="c", subcore_axis_name="s")
    return pl.kernel(
        _dispatch_body,
        out_type=(
            jax.ShapeDtypeStruct((NC, NPLANE, D), jnp.float32),
            jax.ShapeDtypeStruct((NC, NPLANE, 128), jnp.float32),
        ),
        mesh=mesh,
        scratch_types=[
            pltpu.VMEM((CHUNK, D), jnp.float32),
            pltpu.VMEM((CHUNK, D), jnp.float32),
            pltpu.VMEM((CHUNK,), jnp.int32),
            pltpu.VMEM((CHUNK,), jnp.int32),
            pltpu.VMEM((CHUNK, 128), jnp.float32),
            pltpu.VMEM((CHUNK, 128), jnp.float32),
            pltpu.SemaphoreType.DMA,
        ],
    )(x_flat, dsts)


def _ff_body(big_ref, map_ref, w1_ref, w2_ref, w3_ref, out_ref):
    ma = map_ref[0][:, 0:1] > 0.0
    mb = map_ref[1][:, 0:1] > 0.0
    xg = (jnp.where(ma, big_ref[0], 0.0)
          + jnp.where(mb, big_ref[1], 0.0)).astype(jnp.bfloat16)
    h = (jnp.dot(xg, w2_ref[0].astype(jnp.bfloat16),
                 preferred_element_type=jnp.float32)
         * jnp.dot(xg, w1_ref[0].astype(jnp.bfloat16),
                   preferred_element_type=jnp.float32))
    h = jax.nn.gelu(h)
    out_ref[0] = jnp.dot(h.astype(jnp.bfloat16),
                         w3_ref[0].astype(jnp.bfloat16),
                         preferred_element_type=jnp.float32)


def _ff(big, vmap3, w1, w2, w3):
    return pl.pallas_call(
        _ff_body,
        grid=(E,),
        in_specs=[
            pl.BlockSpec((NC, CAP, D), lambda e: (0, e, 0)),
            pl.BlockSpec((NC, CAP, 128), lambda e: (0, e, 0)),
            pl.BlockSpec((1, D, H), lambda e: (e, 0, 0)),
            pl.BlockSpec((1, D, H), lambda e: (e, 0, 0)),
            pl.BlockSpec((1, H, D), lambda e: (e, 0, 0)),
        ],
        out_specs=pl.BlockSpec((1, CAP, D), lambda e: (0, e, 0)),
        out_shape=jax.ShapeDtypeStruct((NC, NPLANE, D), jnp.float32),
        input_output_aliases={0: 0},
    )(big, vmap3, w1, w2, w3)


def _combine_body(big_hbm, src_hbm, g0_hbm, g1_hbm,
                  idx_a, idx_b, rows_a, rows_b, sem_a, sem_b):
    wid = lax.axis_index("s") * NC + lax.axis_index("c")
    base = wid * CHUNK
    pltpu.sync_copy(src_hbm.at[wid, pl.ds(0, CHUNK)], idx_a)
    pltpu.sync_copy(src_hbm.at[wid, pl.ds(CHUNK, CHUNK)], idx_b)
    cp_a = pltpu.async_copy(big_hbm.at[idx_a], rows_a, sem_a)
    cp_b = pltpu.async_copy(big_hbm.at[idx_b], rows_b, sem_b)
    cp_a.wait()
    pltpu.sync_copy(rows_a, g0_hbm.at[pl.ds(base, CHUNK)])
    cp_b.wait()
    pltpu.sync_copy(rows_b, g1_hbm.at[pl.ds(base, CHUNK)])


def _combine(big_flat, srcs2):
    mesh = plsc.VectorSubcoreMesh(core_axis_name="c", subcore_axis_name="s")
    return pl.kernel(
        _combine_body,
        out_type=(
            jax.ShapeDtypeStruct((T, D), jnp.float32),
            jax.ShapeDtypeStruct((T, D), jnp.float32),
        ),
        mesh=mesh,
        scratch_types=[
            pltpu.VMEM((CHUNK,), jnp.int32),
            pltpu.VMEM((CHUNK,), jnp.int32),
            pltpu.VMEM((CHUNK, D), jnp.float32),
            pltpu.VMEM((CHUNK, D), jnp.float32),
            pltpu.SemaphoreType.DMA,
            pltpu.SemaphoreType.DMA,
        ],
    )(big_flat, srcs2)


def _blend_body(g0_ref, g1_ref, x_ref, coef_ref, out_ref):
    c0 = coef_ref[:, 0:1]
    c1 = coef_ref[:, 1:2]
    cx = coef_ref[:, 2:3]
    out_ref[...] = c0 * g0_ref[...] + c1 * g1_ref[...] + cx * x_ref[...]


def _blend(g0, g1, x_flat, coef):
    return pl.pallas_call(
        _blend_body,
        out_shape=jax.ShapeDtypeStruct((T, D), jnp.float32),
    )(g0, g1, x_flat, coef)


def kernel(x, Wr, br, w1, w2, w3):
    x_flat = x.reshape(T, D)
    coef, idx = _router(x_flat, Wr, br.reshape(1, E))
    dsts = idx[:, 0:2].T.copy()
    src0 = idx[:, 2].reshape(NW, CHUNK)
    src1 = idx[:, 3].reshape(NW, CHUNK)
    srcs2 = jnp.concatenate([src0, src1], axis=1)
    big, vmap = _dispatch(x_flat, dsts)
    big = _ff(big, vmap, w1, w2, w3)
    g0, g1 = _combine(big.reshape(NC * NPLANE, D), srcs2)
    out = _blend(g0, g1, x_flat, coef)
    return out.reshape(1, T, D)


# speedup vs baseline: 2.8921x; 1.0966x over previous
import jax
import jax.numpy as jnp
from jax import lax
from jax.experimental import pallas as pl
from jax.experimental.pallas import tpu as pltpu
from jax.experimental.pallas import tpu_sc as plsc

D = 768
H = 1024
E = 8
T = 2048
CAP = 512
NSLOT = E * CAP
NC, NS = 2, 16
NW = NC * NS
NPLANE = NSLOT + CAP
NBIG = 4 * NPLANE
XOFF = 3 * NPLANE
TPS = T // NS
CHUNK = 64
SPS = NSLOT // NS
BLK = 128


def _router_body(x_ref, wr_ref, br_ref, coef_ref, idx_ref):
    x = x_ref[...]
    logits = jnp.dot(x, wr_ref[...], preferred_element_type=jnp.float32)
    logits = logits + br_ref[...]
    eidx = lax.broadcasted_iota(jnp.int32, (T, E), 1)

    m1 = jnp.max(logits, axis=1, keepdims=True)
    a1 = jnp.min(jnp.where(logits == m1, eidx, E), axis=1, keepdims=True)
    oh0 = eidx == a1
    masked = jnp.where(oh0, -jnp.inf, logits)
    m2 = jnp.max(masked, axis=1, keepdims=True)
    a2 = jnp.min(jnp.where(masked == m2, eidx, E), axis=1, keepdims=True)
    oh1 = eidx == a2

    dexp = jnp.exp(m2 - m1)
    s0 = 1.0 / (1.0 + dexp)
    s1 = dexp / (1.0 + dexp)

    oh0f = oh0.astype(jnp.float32)
    oh1f = oh1.astype(jnp.float32)
    both = jnp.concatenate([oh0f, oh0f + oh1f], axis=1)
    r = lax.broadcasted_iota(jnp.int32, (BLK, BLK), 0)
    c = lax.broadcasted_iota(jnp.int32, (BLK, BLK), 1)
    tri = (c <= r).astype(jnp.float32)
    run = jnp.zeros((1, 2 * E), jnp.float32)
    pieces = []
    for b in range(T // BLK):
        blk = both[b * BLK:(b + 1) * BLK, :]
        wcs = jnp.dot(tri, blk, preferred_element_type=jnp.float32) + run
        run = wcs[BLK - 1:BLK, :]
        pieces.append(wcs)
    csum = jnp.concatenate(pieces, axis=0)

    pos0 = jnp.sum(oh0f * csum[:, :E], axis=1, keepdims=True)
    pos1 = jnp.sum(oh1f * csum[:, E:], axis=1, keepdims=True)
    keep0 = pos0 < CAP
    keep1 = pos1 < CAP

    coef_ref[...] = jnp.concatenate(
        [jnp.broadcast_to(s0, (T, 16)), jnp.broadcast_to(s1, (T, 16))],
        axis=1)

    pos0i = pos0.astype(jnp.int32)
    pos1i = pos1.astype(jnp.int32)
    slot0 = a1 * CAP + pos0i
    slot1 = a2 * CAP + pos1i
    tidx = lax.broadcasted_iota(jnp.int32, (T, 1), 0)
    trash = NSLOT + jnp.right_shift(tidx, 7)
    dst0 = jnp.where(keep0, slot0, trash)
    dst1 = jnp.where(keep1, slot1, trash) + NPLANE
    xrow = XOFF + tidx
    src0 = jnp.where(keep0, slot0, xrow)
    src1 = jnp.where(keep1, slot1, xrow)
    zi = jnp.zeros((T, 1), jnp.int32)
    idx_ref[...] = jnp.concatenate(
        [dst0, dst1, src0, src1, pos0i, pos1i, zi, zi], axis=1)


def _router(x_flat, Wr, br2d):
    return pl.pallas_call(
        _router_body,
        out_shape=(
            jax.ShapeDtypeStruct((T, 32), jnp.float32),
            jax.ShapeDtypeStruct((T, 8), jnp.int32),
        ),
    )(x_flat, Wr, br2d)


def _dispatch_body(x_hbm, dst_hbm, big, vmap,
                   rows_a, rows_b, idx_a, idx_b, zv, ones_v, sem):
    cid = lax.axis_index("c")
    sid = lax.axis_index("s")
    def _fill(i, _):
        for j in range(128 // 16):
            zv[i, pl.ds(16 * j, 16)] = jnp.zeros((16,), jnp.float32)
            ones_v[i, pl.ds(16 * j, 16)] = jnp.ones((16,), jnp.float32)
        return 0
    lax.fori_loop(0, CHUNK, _fill, 0)
    for j in range(SPS // CHUNK):
        pltpu.sync_copy(
            zv, vmap.at[pl.ds(cid * NPLANE + sid * SPS + j * CHUNK, CHUNK)])
    plsc.subcore_barrier()
    tb = sid * TPS
    pltpu.sync_copy(dst_hbm.at[cid, pl.ds(tb, CHUNK)], idx_a)
    pltpu.sync_copy(dst_hbm.at[cid, pl.ds(tb + CHUNK, CHUNK)], idx_b)
    pltpu.sync_copy(x_hbm.at[pl.ds(tb, CHUNK)], rows_a)
    cps = [pltpu.async_copy(rows_a, big.at[idx_a], sem),
           pltpu.async_copy(ones_v, vmap.at[idx_a], sem)]
    pltpu.sync_copy(x_hbm.at[pl.ds(tb + CHUNK, CHUNK)], rows_b)
    cps += [pltpu.async_copy(rows_b, big.at[idx_b], sem),
            pltpu.async_copy(ones_v, vmap.at[idx_b], sem)]
    @pl.when(cid == 0)
    def _xcopy():
        cpx0 = pltpu.async_copy(rows_a, big.at[pl.ds(XOFF + tb, CHUNK)], sem)
        cpx1 = pltpu.async_copy(rows_b,
                                big.at[pl.ds(XOFF + tb + CHUNK, CHUNK)], sem)
        cpx0.wait()
        cpx1.wait()
    for cp in cps:
        cp.wait()


def _dispatch(x_flat, dsts):
    mesh = plsc.VectorSubcoreMesh(core_axis_name="c", subcore_axis_name="s")
    return pl.kernel(
        _dispatch_body,
        out_type=(
            jax.ShapeDtypeStruct((NBIG, D), jnp.float32),
            jax.ShapeDtypeStruct((2 * NPLANE, 128), jnp.float32),
        ),
        mesh=mesh,
        scratch_types=[
            pltpu.VMEM((CHUNK, D), jnp.float32),
            pltpu.VMEM((CHUNK, D), jnp.float32),
            pltpu.VMEM((CHUNK,), jnp.int32),
            pltpu.VMEM((CHUNK,), jnp.int32),
            pltpu.VMEM((CHUNK, 128), jnp.float32),
            pltpu.VMEM((CHUNK, 128), jnp.float32),
            pltpu.SemaphoreType.DMA,
        ],
    )(x_flat, dsts)


def _ff_body(big_ref, map_ref, w1_ref, w2_ref, w3_ref, out_ref):
    ma = map_ref[0][:, 0:1] > 0.0
    mb = map_ref[1][:, 0:1] > 0.0
    xg = (jnp.where(ma, big_ref[0], 0.0)
          + jnp.where(mb, big_ref[1], 0.0)).astype(jnp.bfloat16)
    h = (jnp.dot(xg, w2_ref[0].astype(jnp.bfloat16),
                 preferred_element_type=jnp.float32)
         * jnp.dot(xg, w1_ref[0].astype(jnp.bfloat16),
                   preferred_element_type=jnp.float32))
    h = jax.nn.gelu(h)
    out_ref[0] = jnp.dot(h.astype(jnp.bfloat16),
                         w3_ref[0].astype(jnp.bfloat16),
                         preferred_element_type=jnp.float32)


def _ff(big4, vmap3, w1, w2, w3):
    return pl.pallas_call(
        _ff_body,
        grid=(E,),
        in_specs=[
            pl.BlockSpec((2, CAP, D), lambda e: (0, e, 0)),
            pl.BlockSpec((2, CAP, 128), lambda e: (0, e, 0)),
            pl.BlockSpec((1, D, H), lambda e: (e, 0, 0)),
            pl.BlockSpec((1, D, H), lambda e: (e, 0, 0)),
            pl.BlockSpec((1, H, D), lambda e: (e, 0, 0)),
        ],
        out_specs=pl.BlockSpec((1, CAP, D), lambda e: (0, e, 0)),
        out_shape=jax.ShapeDtypeStruct((4, NPLANE, D), jnp.float32),
        input_output_aliases={0: 0},
    )(big4, vmap3, w1, w2, w3)


def _combine_body(big_hbm, src_hbm, coef_hbm, out_hbm,
                  idx_a, idx_b, rows_a, rows_b, coef_v, sem_a, sem_b):
    wid = lax.axis_index("s") * NC + lax.axis_index("c")
    base = wid * CHUNK
    pltpu.sync_copy(src_hbm.at[wid, pl.ds(0, CHUNK)], idx_a)
    pltpu.sync_copy(src_hbm.at[wid, pl.ds(CHUNK, CHUNK)], idx_b)
    cp_a = pltpu.async_copy(big_hbm.at[idx_a], rows_a, sem_a)
    cp_b = pltpu.async_copy(big_hbm.at[idx_b], rows_b, sem_b)
    pltpu.sync_copy(coef_hbm.at[pl.ds(base, CHUNK)], coef_v)
    cp_a.wait()
    cp_b.wait()

    def _tok(i, _):
        s0 = coef_v[i, pl.ds(0, 16)]
        s1 = coef_v[i, pl.ds(16, 16)]
        for j in range(D // 16):
            ga = rows_a[i, pl.ds(16 * j, 16)]
            gb = rows_b[i, pl.ds(16 * j, 16)]
            rows_a[i, pl.ds(16 * j, 16)] = s0 * ga + s1 * gb
        return 0
    lax.fori_loop(0, CHUNK, _tok, 0)
    pltpu.sync_copy(rows_a, out_hbm.at[pl.ds(base, CHUNK)])


def _combine(big_flat, srcs2, coef):
    mesh = plsc.VectorSubcoreMesh(core_axis_name="c", subcore_axis_name="s")
    return pl.kernel(
        _combine_body,
        out_type=jax.ShapeDtypeStruct((T, D), jnp.float32),
        mesh=mesh,
        scratch_types=[
            pltpu.VMEM((CHUNK,), jnp.int32),
            pltpu.VMEM((CHUNK,), jnp.int32),
            pltpu.VMEM((CHUNK, D), jnp.float32),
            pltpu.VMEM((CHUNK, D), jnp.float32),
            pltpu.VMEM((CHUNK, 32), jnp.float32),
            pltpu.SemaphoreType.DMA,
            pltpu.SemaphoreType.DMA,
        ],
    )(big_flat, srcs2, coef)


def kernel(x, Wr, br, w1, w2, w3):
    x_flat = x.reshape(T, D)
    coef, idx = _router(x_flat, Wr, br.reshape(1, E))
    dsts = idx[:, 0:2].T.copy()
    src0 = idx[:, 2].reshape(NW, CHUNK)
    src1 = idx[:, 3].reshape(NW, CHUNK)
    srcs2 = jnp.concatenate([src0, src1], axis=1)
    big, vmap = _dispatch(x_flat, dsts)
    big = _ff(big.reshape(4, NPLANE, D), vmap.reshape(2, NPLANE, 128),
              w1, w2, w3)
    out = _combine(big.reshape(NBIG, D), srcs2, coef)
    return out.reshape(1, T, D)


# speedup vs baseline: 2.9646x; 1.0251x over previous
import jax
import jax.numpy as jnp
from jax import lax
from jax.experimental import pallas as pl
from jax.experimental.pallas import tpu as pltpu
from jax.experimental.pallas import tpu_sc as plsc

D = 768
H = 1024
E = 8
T = 2048
CAP = 512
NSLOT = E * CAP
NC, NS = 2, 16
NW = NC * NS
NPLANE = NSLOT + CAP
NBIG = 4 * NPLANE
XOFF = 3 * NPLANE
TPS = T // NS
CHUNK = 64
SPS = NSLOT // NS
BLK = 128


def _router_body(x_ref, wr_ref, br_ref, coef_ref, idx_ref):
    x = x_ref[...]
    logits = jnp.dot(x, wr_ref[...], preferred_element_type=jnp.float32)
    logits = logits + br_ref[...]
    eidx = lax.broadcasted_iota(jnp.int32, (T, E), 1)

    m1 = jnp.max(logits, axis=1, keepdims=True)
    a1 = jnp.min(jnp.where(logits == m1, eidx, E), axis=1, keepdims=True)
    oh0 = eidx == a1
    masked = jnp.where(oh0, -jnp.inf, logits)
    m2 = jnp.max(masked, axis=1, keepdims=True)
    a2 = jnp.min(jnp.where(masked == m2, eidx, E), axis=1, keepdims=True)
    oh1 = eidx == a2

    dexp = jnp.exp(m2 - m1)
    s0 = 1.0 / (1.0 + dexp)
    s1 = dexp / (1.0 + dexp)

    oh0f = oh0.astype(jnp.float32)
    oh1f = oh1.astype(jnp.float32)
    both = jnp.concatenate([oh0f, oh0f + oh1f], axis=1)
    r = lax.broadcasted_iota(jnp.int32, (BLK, BLK), 0)
    c = lax.broadcasted_iota(jnp.int32, (BLK, BLK), 1)
    tri = (c <= r).astype(jnp.float32)
    run = jnp.zeros((1, 2 * E), jnp.float32)
    pieces = []
    for b in range(T // BLK):
        blk = both[b * BLK:(b + 1) * BLK, :]
        wcs = jnp.dot(tri, blk, preferred_element_type=jnp.float32) + run
        run = wcs[BLK - 1:BLK, :]
        pieces.append(wcs)
    csum = jnp.concatenate(pieces, axis=0)

    pos0 = jnp.sum(oh0f * csum[:, :E], axis=1, keepdims=True)
    pos1 = jnp.sum(oh1f * csum[:, E:], axis=1, keepdims=True)
    keep0 = pos0 < CAP
    keep1 = pos1 < CAP

    coef_ref[...] = jnp.concatenate(
        [jnp.broadcast_to(s0, (T, 16)), jnp.broadcast_to(s1, (T, 16))],
        axis=1)

    pos0i = pos0.astype(jnp.int32)
    pos1i = pos1.astype(jnp.int32)
    slot0 = a1 * CAP + pos0i
    slot1 = a2 * CAP + pos1i
    tidx = lax.broadcasted_iota(jnp.int32, (T, 1), 0)
    trash = NSLOT + jnp.right_shift(tidx, 7)
    dst0 = jnp.where(keep0, slot0, trash)
    dst1 = jnp.where(keep1, slot1, trash) + NPLANE
    xrow = XOFF + tidx
    src0 = jnp.where(keep0, slot0, xrow)
    src1 = jnp.where(keep1, slot1, xrow)
    zi = jnp.zeros((T, 1), jnp.int32)
    idx_ref[...] = jnp.concatenate(
        [dst0, dst1, src0, src1, pos0i, pos1i, zi, zi], axis=1)


def _router(x_flat, Wr, br2d):
    return pl.pallas_call(
        _router_body,
        out_shape=(
            jax.ShapeDtypeStruct((T, 32), jnp.float32),
            jax.ShapeDtypeStruct((T, 8), jnp.int32),
        ),
    )(x_flat, Wr, br2d)


def _dispatch_body(x_hbm, dst_hbm, big, vmap,
                   rows_a, rows_b, idx_a, idx_b, zv, ones_v, sem, sem_in):
    cid = lax.axis_index("c")
    sid = lax.axis_index("s")
    tb = sid * TPS
    ld = [pltpu.async_copy(dst_hbm.at[cid, pl.ds(tb, CHUNK)], idx_a, sem_in),
          pltpu.async_copy(dst_hbm.at[cid, pl.ds(tb + CHUNK, CHUNK)], idx_b,
                           sem_in),
          pltpu.async_copy(x_hbm.at[pl.ds(tb, CHUNK)], rows_a, sem_in),
          pltpu.async_copy(x_hbm.at[pl.ds(tb + CHUNK, CHUNK)], rows_b,
                           sem_in)]
    def _fill(i, _):
        for j in range(128 // 16):
            zv[i, pl.ds(16 * j, 16)] = jnp.zeros((16,), jnp.float32)
            ones_v[i, pl.ds(16 * j, 16)] = jnp.ones((16,), jnp.float32)
        return 0
    lax.fori_loop(0, CHUNK, _fill, 0)
    zcps = [pltpu.async_copy(
        zv, vmap.at[pl.ds(cid * NPLANE + sid * SPS + j * CHUNK, CHUNK)], sem)
        for j in range(SPS // CHUNK)]
    for cp in zcps:
        cp.wait()
    plsc.subcore_barrier()
    for cp in ld:
        cp.wait()
    cps = [pltpu.async_copy(rows_a, big.at[idx_a], sem),
           pltpu.async_copy(ones_v, vmap.at[idx_a], sem),
           pltpu.async_copy(rows_b, big.at[idx_b], sem),
           pltpu.async_copy(ones_v, vmap.at[idx_b], sem)]
    @pl.when(cid == 0)
    def _xcopy():
        cpx0 = pltpu.async_copy(rows_a, big.at[pl.ds(XOFF + tb, CHUNK)], sem)
        cpx1 = pltpu.async_copy(rows_b,
                                big.at[pl.ds(XOFF + tb + CHUNK, CHUNK)], sem)
        cpx0.wait()
        cpx1.wait()
    for cp in cps:
        cp.wait()


def _dispatch(x_flat, dsts):
    mesh = plsc.VectorSubcoreMesh(core_axis_name="c", subcore_axis_name="s")
    return pl.kernel(
        _dispatch_body,
        out_type=(
            jax.ShapeDtypeStruct((NBIG, D), jnp.float32),
            jax.ShapeDtypeStruct((2 * NPLANE, 128), jnp.float32),
        ),
        mesh=mesh,
        scratch_types=[
            pltpu.VMEM((CHUNK, D), jnp.float32),
            pltpu.VMEM((CHUNK, D), jnp.float32),
            pltpu.VMEM((CHUNK,), jnp.int32),
            pltpu.VMEM((CHUNK,), jnp.int32),
            pltpu.VMEM((CHUNK, 128), jnp.float32),
            pltpu.VMEM((CHUNK, 128), jnp.float32),
            pltpu.SemaphoreType.DMA,
            pltpu.SemaphoreType.DMA,
        ],
    )(x_flat, dsts)


def _ff_body(big_ref, map_ref, w1_ref, w2_ref, w3_ref, out_ref):
    ma = map_ref[0][:, 0:1] > 0.0
    mb = map_ref[1][:, 0:1] > 0.0
    xg = (jnp.where(ma, big_ref[0], 0.0)
          + jnp.where(mb, big_ref[1], 0.0)).astype(jnp.bfloat16)
    h = (jnp.dot(xg, w2_ref[0].astype(jnp.bfloat16),
                 preferred_element_type=jnp.float32)
         * jnp.dot(xg, w1_ref[0].astype(jnp.bfloat16),
                   preferred_element_type=jnp.float32))
    h = jax.nn.gelu(h)
    out_ref[0] = jnp.dot(h.astype(jnp.bfloat16),
                         w3_ref[0].astype(jnp.bfloat16),
                         preferred_element_type=jnp.float32)


def _ff(big4, vmap3, w1, w2, w3):
    return pl.pallas_call(
        _ff_body,
        grid=(E,),
        in_specs=[
            pl.BlockSpec((2, CAP, D), lambda e: (0, e, 0)),
            pl.BlockSpec((2, CAP, 128), lambda e: (0, e, 0)),
            pl.BlockSpec((1, D, H), lambda e: (e, 0, 0)),
            pl.BlockSpec((1, D, H), lambda e: (e, 0, 0)),
            pl.BlockSpec((1, H, D), lambda e: (e, 0, 0)),
        ],
        out_specs=pl.BlockSpec((1, CAP, D), lambda e: (0, e, 0)),
        out_shape=jax.ShapeDtypeStruct((4, NPLANE, D), jnp.float32),
        input_output_aliases={0: 0},
    )(big4, vmap3, w1, w2, w3)


def _combine_body(big_hbm, src_hbm, coef_hbm, out_hbm,
                  idx_a, idx_b, rows_a, rows_b, coef_v, sem_a, sem_b):
    wid = lax.axis_index("s") * NC + lax.axis_index("c")
    base = wid * CHUNK
    pltpu.sync_copy(src_hbm.at[wid, pl.ds(0, CHUNK)], idx_a)
    pltpu.sync_copy(src_hbm.at[wid, pl.ds(CHUNK, CHUNK)], idx_b)
    cp_a = pltpu.async_copy(big_hbm.at[idx_a], rows_a, sem_a)
    cp_b = pltpu.async_copy(big_hbm.at[idx_b], rows_b, sem_b)
    pltpu.sync_copy(coef_hbm.at[pl.ds(base, CHUNK)], coef_v)
    cp_a.wait()
    cp_b.wait()

    def _tok(i, _):
        s0 = coef_v[i, pl.ds(0, 16)]
        s1 = coef_v[i, pl.ds(16, 16)]
        for j in range(D // 16):
            ga = rows_a[i, pl.ds(16 * j, 16)]
            gb = rows_b[i, pl.ds(16 * j, 16)]
            rows_a[i, pl.ds(16 * j, 16)] = s0 * ga + s1 * gb
        return 0
    lax.fori_loop(0, CHUNK, _tok, 0)
    pltpu.sync_copy(rows_a, out_hbm.at[pl.ds(base, CHUNK)])


def _combine(big_flat, srcs2, coef):
    mesh = plsc.VectorSubcoreMesh(core_axis_name="c", subcore_axis_name="s")
    return pl.kernel(
        _combine_body,
        out_type=jax.ShapeDtypeStruct((T, D), jnp.float32),
        mesh=mesh,
        scratch_types=[
            pltpu.VMEM((CHUNK,), jnp.int32),
            pltpu.VMEM((CHUNK,), jnp.int32),
            pltpu.VMEM((CHUNK, D), jnp.float32),
            pltpu.VMEM((CHUNK, D), jnp.float32),
            pltpu.VMEM((CHUNK, 32), jnp.float32),
            pltpu.SemaphoreType.DMA,
            pltpu.SemaphoreType.DMA,
        ],
    )(big_flat, srcs2, coef)


def kernel(x, Wr, br, w1, w2, w3):
    x_flat = x.reshape(T, D)
    coef, idx = _router(x_flat, Wr, br.reshape(1, E))
    dsts = idx[:, 0:2].T.copy()
    src0 = idx[:, 2].reshape(NW, CHUNK)
    src1 = idx[:, 3].reshape(NW, CHUNK)
    srcs2 = jnp.concatenate([src0, src1], axis=1)
    big, vmap = _dispatch(x_flat, dsts)
    big = _ff(big.reshape(4, NPLANE, D), vmap.reshape(2, NPLANE, 128),
              w1, w2, w3)
    out = _combine(big.reshape(NBIG, D), srcs2, coef)
    return out.reshape(1, T, D)
